# Initial kernel scaffold; baseline (speedup 1.0000x reference)
import jax
import jax.numpy as jnp
from jax.experimental import pallas as pl


def kernel(japan_tokens, token_embedding):
    raise NotImplementedError("write your pallas kernel here")


# speedup vs baseline: 3.4285x; 3.4285x over previous
import jax
import jax.numpy as jnp
from jax.experimental import pallas as pl
from jax.experimental.pallas import tpu as pltpu
from jax.experimental.pallas import tpu_sc as plsc

_B = 4096
_S = 200
_V = 1000
_D = 64
_N = _B * _S
_W = 128


def _pos_encoding():
    even = jnp.arange(0, _D, 2).astype(jnp.float32)
    denom = jnp.power(10000.0, even / _D)
    pos = jnp.arange(_S).reshape(-1, 1).astype(jnp.float32)
    even_pe = jnp.sin(pos / denom)
    odd_pe = jnp.cos(pos / denom)
    return jnp.stack([even_pe, odd_pe], axis=2).reshape(_S, _D)


def _build_big(table, pe):
    vb = 40

    def body(t_ref, p_ref, o_ref):
        o_ref[...] = t_ref[...][:, None, :] + p_ref[...][None, :, :]

    return pl.pallas_call(
        body,
        grid=(_V // vb,),
        in_specs=[
            pl.BlockSpec((vb, _D), lambda i: (i, 0)),
            pl.BlockSpec((_S, _D), lambda i: (0, 0)),
        ],
        out_specs=pl.BlockSpec((vb, _S, _D), lambda i: (i, 0, 0)),
        out_shape=jax.ShapeDtypeStruct((_V, _S, _D), jnp.float32),
    )(table, pe)


def _build_idx(tokens):
    bb = 512

    def body(t_ref, o_ref):
        o_ref[...] = t_ref[...] * _S + jax.lax.broadcasted_iota(
            jnp.int32, (bb, _S), 1)

    return pl.pallas_call(
        body,
        grid=(_B // bb,),
        in_specs=[pl.BlockSpec((bb, _S), lambda i: (i, 0))],
        out_specs=pl.BlockSpec((bb, _S), lambda i: (i, 0)),
        out_shape=jax.ShapeDtypeStruct((_B, _S), jnp.int32),
    )(tokens)


def _sc_gather(big, idx):
    mesh = plsc.VectorSubcoreMesh(core_axis_name="core",
                                  subcore_axis_name="subcore")

    @pl.kernel(out_type=jax.ShapeDtypeStruct((_N, _D), jnp.float32),
               mesh=mesh,
               compiler_params=pltpu.CompilerParams(use_tc_tiling_on_sc=False))
    def k(big_hbm, idx_hbm, o_hbm):
        def body(i_vmem, o_vmem):
            pltpu.sync_copy(big_hbm.at[i_vmem.at[0]], o_vmem)

        pltpu.emit_pipeline(
            body,
            grid=(_N // _W,),
            in_specs=[pl.BlockSpec((1, _W), lambda i: (0, i))],
            out_specs=[pl.BlockSpec((_W, _D), lambda i: (i, 0))],
            core_axis_name=("core", "subcore"),
            dimension_semantics=(pltpu.PARALLEL,),
        )(idx_hbm, o_hbm)

    return k(big, idx)


def kernel(japan_tokens, token_embedding):
    tokens = japan_tokens.astype(jnp.int32)
    pe = _pos_encoding()
    big = _build_big(token_embedding, pe).reshape(_V * _S, _D)
    idx = _build_idx(tokens).reshape(1, _N)
    out = _sc_gather(big, idx)
    return out.reshape(_B, _S, _D)


# speedup vs baseline: 3.7792x; 1.1023x over previous
import jax
import jax.numpy as jnp
from jax.experimental import pallas as pl
from jax.experimental.pallas import tpu as pltpu
from jax.experimental.pallas import tpu_sc as plsc

_B = 4096
_S = 200
_V = 1000
_D = 64
_N = _B * _S
_W = 128


def _pos_encoding():
    even = jnp.arange(0, _D, 2).astype(jnp.float32)
    denom = jnp.power(10000.0, even / _D)
    pos = jnp.arange(_S).reshape(-1, 1).astype(jnp.float32)
    even_pe = jnp.sin(pos / denom)
    odd_pe = jnp.cos(pos / denom)
    return jnp.stack([even_pe, odd_pe], axis=2).reshape(_S, _D)


def _build_big(table, pe):
    vb = 40
    pe128 = pe.reshape(_S // 2, 2 * _D)

    def body(t_ref, p_ref, o_ref):
        t2 = jnp.concatenate([t_ref[...], t_ref[...]], axis=-1)
        o_ref[...] = (t2[:, None, :] + p_ref[...][None, :, :]).reshape(
            vb * (_S // 2), 2 * _D)

    return pl.pallas_call(
        body,
        grid=(_V // vb,),
        in_specs=[
            pl.BlockSpec((vb, _D), lambda i: (i, 0)),
            pl.BlockSpec((_S // 2, 2 * _D), lambda i: (0, 0)),
        ],
        out_specs=pl.BlockSpec((vb * (_S // 2), 2 * _D), lambda i: (i, 0)),
        out_shape=jax.ShapeDtypeStruct((_V * (_S // 2), 2 * _D), jnp.float32),
    )(table, pe128)


def _build_idx(tokens):
    bb = 512

    def body(t_ref, o_ref):
        o_ref[...] = t_ref[...] * _S + jax.lax.broadcasted_iota(
            jnp.int32, (bb, _S), 1)

    return pl.pallas_call(
        body,
        grid=(_B // bb,),
        in_specs=[pl.BlockSpec((bb, _S), lambda i: (i, 0))],
        out_specs=pl.BlockSpec((bb, _S), lambda i: (i, 0)),
        out_shape=jax.ShapeDtypeStruct((_B, _S), jnp.int32),
    )(tokens)


def _sc_gather(big, idx):
    mesh = plsc.VectorSubcoreMesh(core_axis_name="core",
                                  subcore_axis_name="subcore")

    @pl.kernel(out_type=jax.ShapeDtypeStruct((_N, _D), jnp.float32),
               mesh=mesh,
               compiler_params=pltpu.CompilerParams(use_tc_tiling_on_sc=False))
    def k(big_hbm, idx_hbm, o_hbm):
        def body(i_vmem, o_vmem):
            pltpu.sync_copy(big_hbm.at[i_vmem.at[0]], o_vmem)

        pltpu.emit_pipeline(
            body,
            grid=(_N // _W,),
            in_specs=[pl.BlockSpec((1, _W), lambda i: (0, i))],
            out_specs=[pl.BlockSpec((_W, _D), lambda i: (i, 0))],
            core_axis_name=("core", "subcore"),
            dimension_semantics=(pltpu.PARALLEL,),
        )(idx_hbm, o_hbm)

    return k(big, idx)


def kernel(japan_tokens, token_embedding):
    tokens = japan_tokens.astype(jnp.int32)
    pe = _pos_encoding()
    big = _build_big(token_embedding, pe).reshape(_V * _S, _D)
    idx = _build_idx(tokens).reshape(1, _N)
    out = _sc_gather(big, idx)
    return out.reshape(_B, _S, _D)


# speedup vs baseline: 5.1034x; 1.3504x over previous
import jax
import jax.numpy as jnp
from jax.experimental import pallas as pl
from jax.experimental.pallas import tpu as pltpu
from jax.experimental.pallas import tpu_sc as plsc

_B = 4096
_S = 200
_V = 1000
_D = 64
_N = _B * _S
_W = 128
_H = _B // 2


def _pos_encoding():
    even = jnp.arange(0, _D, 2).astype(jnp.float32)
    denom = jnp.power(10000.0, even / _D)
    pos = jnp.arange(_S).reshape(-1, 1).astype(jnp.float32)
    even_pe = jnp.sin(pos / denom)
    odd_pe = jnp.cos(pos / denom)
    return jnp.stack([even_pe, odd_pe], axis=2).reshape(_S, _D)


def _build_big(table, pe):
    vb = 40
    pe128 = pe.reshape(_S // 2, 2 * _D)

    def body(t_ref, p_ref, o_ref):
        t2 = jnp.concatenate([t_ref[...], t_ref[...]], axis=-1)
        o_ref[...] = (t2[:, None, :] + p_ref[...][None, :, :]).reshape(
            vb * (_S // 2), 2 * _D)

    return pl.pallas_call(
        body,
        grid=(_V // vb,),
        in_specs=[
            pl.BlockSpec((vb, _D), lambda i: (i, 0)),
            pl.BlockSpec((_S // 2, 2 * _D), lambda i: (0, 0)),
        ],
        out_specs=pl.BlockSpec((vb * (_S // 2), 2 * _D), lambda i: (i, 0)),
        out_shape=jax.ShapeDtypeStruct((_V * (_S // 2), 2 * _D), jnp.float32),
        compiler_params=pltpu.CompilerParams(
            dimension_semantics=("parallel",)),
    )(table, pe128)


def _build_idx(tokens_t):
    sb = 40

    def body(t_ref, o_ref):
        s = (pl.program_id(0) * sb
             + jax.lax.broadcasted_iota(jnp.int32, (sb, _B), 0))
        o_ref[...] = t_ref[...] * _S + s

    return pl.pallas_call(
        body,
        grid=(_S // sb,),
        in_specs=[pl.BlockSpec((sb, _B), lambda i: (i, 0))],
        out_specs=pl.BlockSpec((sb, _B), lambda i: (i, 0)),
        out_shape=jax.ShapeDtypeStruct((_S, _B), jnp.int32),
        compiler_params=pltpu.CompilerParams(
            dimension_semantics=("parallel",)),
    )(tokens_t)


def _sc_gather(big, idx):
    mesh = plsc.VectorSubcoreMesh(core_axis_name="core",
                                  subcore_axis_name="subcore")

    @pl.kernel(out_type=jax.ShapeDtypeStruct((_N, _D), jnp.float32),
               mesh=mesh,
               compiler_params=pltpu.CompilerParams(use_tc_tiling_on_sc=False))
    def k(big_hbm, idx_hbm, o_hbm):
        def body(i_vmem, o_vmem):
            pltpu.sync_copy(big_hbm.at[i_vmem.at[0]], o_vmem)

        pltpu.emit_pipeline(
            body,
            grid=(_N // _W,),
            in_specs=[pl.BlockSpec((1, _W), lambda i: (0, i))],
            out_specs=[pl.BlockSpec((_W, _D), lambda i: (i, 0))],
            core_axis_name=("core", "subcore"),
            dimension_semantics=(pltpu.PARALLEL,),
        )(idx_hbm, o_hbm)

    return k(big, idx)


def _relayout(g):
    def body(g_ref, o_ref):
        x = g_ref[0]
        y0 = x[:, :_D].T
        y1 = x[:, _D:].T
        o_ref[...] = jnp.concatenate([y0, y1], axis=-1)[None]

    return pl.pallas_call(
        body,
        grid=(_S,),
        in_specs=[pl.BlockSpec((1, _H, 2 * _D), lambda i: (i, 0, 0))],
        out_specs=pl.BlockSpec((1, _D, _B), lambda i: (i, 0, 0)),
        out_shape=jax.ShapeDtypeStruct((_S, _D, _B), jnp.float32),
        compiler_params=pltpu.CompilerParams(
            dimension_semantics=("parallel",)),
    )(g)


def kernel(japan_tokens, token_embedding):
    tokens_t = (japan_tokens.astype(jnp.int32).T
                .reshape(_S, 2, _H).transpose(0, 2, 1).reshape(_S, _B))
    pe = _pos_encoding()
    big = _build_big(token_embedding, pe).reshape(_V * _S, _D)
    idx = _build_idx(tokens_t).reshape(1, _N)
    g = _sc_gather(big, idx)
    out3 = _relayout(g.reshape(_S, _H, 2 * _D))
    return jnp.transpose(out3, (2, 0, 1))


# speedup vs baseline: 5.7162x; 1.1201x over previous
import jax
import jax.numpy as jnp
import numpy as np
from jax.experimental import pallas as pl
from jax.experimental.pallas import tpu as pltpu
from jax.experimental.pallas import tpu_sc as plsc

_B = 4096
_S = 200
_V = 1000
_D = 64
_N = _B * _S
_W = 128
_H = _B // 2


def _pos_encoding_np():
    even = np.arange(0, _D, 2, dtype=np.float64)
    denom = np.power(10000.0, even / _D)
    pos = np.arange(_S, dtype=np.float64).reshape(-1, 1)
    even_pe = np.sin(pos / denom)
    odd_pe = np.cos(pos / denom)
    return np.stack([even_pe, odd_pe], axis=2).reshape(_S, _D).astype(np.float32)


_PE128 = _pos_encoding_np().reshape(_S // 2, 2 * _D)


def _build_big(table):
    vb = 40
    pe128 = jnp.asarray(_PE128)

    def body(t_ref, p_ref, o_ref):
        t2 = jnp.concatenate([t_ref[...], t_ref[...]], axis=-1)
        o_ref[...] = (t2[:, None, :] + p_ref[...][None, :, :]).reshape(
            vb * (_S // 2), 2 * _D)

    return pl.pallas_call(
        body,
        grid=(_V // vb,),
        in_specs=[
            pl.BlockSpec((vb, _D), lambda i: (i, 0)),
            pl.BlockSpec((_S // 2, 2 * _D), lambda i: (0, 0)),
        ],
        out_specs=pl.BlockSpec((vb * (_S // 2), 2 * _D), lambda i: (i, 0)),
        out_shape=jax.ShapeDtypeStruct((_V * (_S // 2), 2 * _D), jnp.float32),
        compiler_params=pltpu.CompilerParams(
            dimension_semantics=("parallel",)),
    )(table, pe128)


def _build_idx(tokens_t):
    sb = 40

    def body(t_ref, o_ref):
        s = (pl.program_id(0) * sb
             + jax.lax.broadcasted_iota(jnp.int32, (sb, _B), 0))
        o_ref[...] = t_ref[...] * _S + s

    return pl.pallas_call(
        body,
        grid=(_S // sb,),
        in_specs=[pl.BlockSpec((sb, _B), lambda i: (i, 0))],
        out_specs=pl.BlockSpec((sb, _B), lambda i: (i, 0)),
        out_shape=jax.ShapeDtypeStruct((_S, _B), jnp.int32),
        compiler_params=pltpu.CompilerParams(
            dimension_semantics=("parallel",)),
    )(tokens_t)


def _sc_gather(big, idx):
    mesh = plsc.VectorSubcoreMesh(core_axis_name="core",
                                  subcore_axis_name="subcore")

    @pl.kernel(out_type=jax.ShapeDtypeStruct((_N, _D), jnp.float32),
               mesh=mesh,
               compiler_params=pltpu.CompilerParams(use_tc_tiling_on_sc=False))
    def k(big_hbm, idx_hbm, o_hbm):
        def body(i_vmem, o_vmem):
            pltpu.sync_copy(big_hbm.at[i_vmem.at[0]], o_vmem)

        pltpu.emit_pipeline(
            body,
            grid=(_N // _W,),
            in_specs=[pl.BlockSpec((1, _W), lambda i: (0, i))],
            out_specs=[pl.BlockSpec((_W, _D), lambda i: (i, 0))],
            core_axis_name=("core", "subcore"),
            dimension_semantics=(pltpu.PARALLEL,),
        )(idx_hbm, o_hbm)

    return k(big, idx)


def _relayout(g):
    sb = 2

    def body(g_ref, o_ref):
        for k in range(sb):
            x = g_ref[k]
            y0 = x[:, :_D].T
            y1 = x[:, _D:].T
            o_ref[k] = jnp.concatenate([y0, y1], axis=-1)

    return pl.pallas_call(
        body,
        grid=(_S // sb,),
        in_specs=[pl.BlockSpec((sb, _H, 2 * _D), lambda i: (i, 0, 0))],
        out_specs=pl.BlockSpec((sb, _D, _B), lambda i: (i, 0, 0)),
        out_shape=jax.ShapeDtypeStruct((_S, _D, _B), jnp.float32),
        compiler_params=pltpu.CompilerParams(
            dimension_semantics=("parallel",)),
    )(g)


def kernel(japan_tokens, token_embedding):
    tokens_t = (japan_tokens.astype(jnp.int32).T
                .reshape(_S, 2, _H).transpose(0, 2, 1).reshape(_S, _B))
    big = _build_big(token_embedding).reshape(_V * _S, _D)
    idx = _build_idx(tokens_t).reshape(1, _N)
    g = _sc_gather(big, idx)
    out3 = _relayout(g.reshape(_S, _H, 2 * _D))
    return jnp.transpose(out3, (2, 0, 1))
